# Initial kernel scaffold; baseline (speedup 1.0000x reference)
import jax
import jax.numpy as jnp
from jax.experimental import pallas as pl


def kernel(x, edge_index, edge_attr, batch, W_gc0, b_gc0, gamma0, beta0, W_gc1, b_gc1, gamma1, beta1, lin1_W, lin1_b, lin2_W, lin2_b, cls1_W, cls1_b, cls2_W, cls2_b):
    raise NotImplementedError("write your pallas kernel here")


# speedup vs baseline: 14.6733x; 14.6733x over previous
import functools

import jax
import jax.numpy as jnp
from jax import lax
from jax.experimental import pallas as pl
from jax.experimental.pallas import tpu as pltpu
from jax.experimental.pallas import tpu_sc as plsc

N = 10000
E = 320000
D = 128
OUT = 16
G = 64
EPS = 1e-5

NC, NS = 2, 16
NW = NC * NS
EW_PER = E // NW
CHUNK = 80
NCHUNK = EW_PER // CHUNK
WB_TILES = 10
WB_ROWS = N // WB_TILES
LANES = 16

_mesh = plsc.VectorSubcoreMesh(core_axis_name="c", subcore_axis_name="s",
                               num_cores=NC, num_subcores=NS)


_sc_params = pltpu.CompilerParams(use_tc_tiling_on_sc=False)


@functools.partial(
    pl.kernel,
    out_type=jax.ShapeDtypeStruct((NC * N,), jnp.float32),
    mesh=_mesh,
    scratch_types=[
        pltpu.VMEM((NCHUNK, CHUNK), jnp.int32),
        pltpu.VMEM((EW_PER,), jnp.float32),
        pltpu.VMEM((2000,), jnp.float32),
        pltpu.VMEM_SHARED((N,), jnp.float32),
    ],
    compiler_params=_sc_params,
)
def _deg_kernel(col_hbm, ew_hbm, out_hbm, col_v, ew_f, zbuf, acc):
    c = lax.axis_index("c")
    s = lax.axis_index("s")
    wid = s * NC + c
    base = wid * EW_PER

    def cfill(j, carry):
        pltpu.sync_copy(col_hbm.at[pl.ds(base + j * CHUNK, CHUNK)],
                        col_v.at[j])
        return carry

    lax.fori_loop(0, NCHUNK, cfill, 0)
    pltpu.sync_copy(ew_hbm.at[pl.ds(base, EW_PER)], ew_f)

    def zf(i, carry):
        zbuf[pl.ds(i * LANES, LANES)] = jnp.zeros((LANES,), jnp.float32)
        return carry

    lax.fori_loop(0, 2000 // LANES, zf, 0)

    @pl.when(s < 5)
    def _():
        pltpu.sync_copy(zbuf, acc.at[pl.ds(s * 2000, 2000)])

    plsc.subcore_barrier()

    def body(j, carry):
        pltpu.sync_copy(ew_f.at[pl.ds(j * CHUNK, CHUNK)],
                        acc.at[col_v.at[j]], add=True)
        return carry

    lax.fori_loop(0, NCHUNK, body, 0)
    plsc.subcore_barrier()

    @pl.when(s < 5)
    def _():
        pltpu.sync_copy(acc.at[pl.ds(s * 2000, 2000)], zbuf)
        pltpu.sync_copy(zbuf, out_hbm.at[pl.ds(c * N + s * 2000, 2000)])


@functools.partial(
    pl.kernel,
    out_type=jax.ShapeDtypeStruct((NC * N, D), jnp.float32),
    mesh=_mesh,
    scratch_types=[
        pltpu.VMEM((NCHUNK + 3, CHUNK), jnp.int32),
        pltpu.VMEM((NCHUNK + 3, CHUNK), jnp.int32),
        pltpu.VMEM((NCHUNK + 3, CHUNK), jnp.float32),
        pltpu.VMEM((CHUNK, D), jnp.float32),
        pltpu.VMEM((8, D), jnp.float32),
        pltpu.VMEM((NCHUNK + 3,), jnp.int32),
        pltpu.VMEM_SHARED((N, D), jnp.float32),
        pltpu.SemaphoreType.DMA,
    ],
    compiler_params=_sc_params,
)
def _scat_kernel(hs_hbm, row_hbm, col_hbm, ew_hbm, out_hbm,
                 row_v, col_v, ew_v, gbuf, zbuf, idx_v, acc, gsem):
    c = lax.axis_index("c")
    s = lax.axis_index("s")
    wid = s * NC + c

    def ifill(i, carry):
        iv = (lax.broadcasted_iota(jnp.int32, (LANES,), 0)
              + (wid * NCHUNK + i * LANES))
        idx_v[pl.ds(i * LANES, LANES)] = jnp.minimum(iv, NW * NCHUNK - 1)
        return carry

    lax.fori_loop(0, (NCHUNK + 3) // LANES, ifill, 0)
    pltpu.async_copy(row_hbm.at[idx_v], row_v, gsem).wait()
    pltpu.async_copy(col_hbm.at[idx_v], col_v, gsem).wait()
    pltpu.async_copy(ew_hbm.at[idx_v], ew_v, gsem).wait()

    def zf(r, carry):
        for cc in range(D // LANES):
            zbuf[r, pl.ds(cc * LANES, LANES)] = jnp.zeros((LANES,), jnp.float32)
        return carry

    lax.fori_loop(0, 8, zf, 0)

    @pl.when(s < WB_TILES)
    def _():
        def zc(m, carry):
            pltpu.sync_copy(zbuf, acc.at[pl.ds(s * WB_ROWS + m * 8, 8)])
            return carry

        lax.fori_loop(0, WB_ROWS // 8, zc, 0)

    plsc.subcore_barrier()

    dn = lax.GatherDimensionNumbers(
        offset_dims=(), collapsed_slice_dims=(0,), start_index_map=(0,))

    def chunk(j, carry):
        pltpu.async_copy(hs_hbm.at[row_v.at[j]], gbuf, gsem).wait()

        def scale16(rb, carry2):
            wch = ew_v[j, pl.ds(rb * LANES, LANES)]

            def lane(l, carry3):
                wv = lax.gather(wch, jnp.full((LANES, 1), l, jnp.int32), dn,
                                slice_sizes=(1,),
                                mode=lax.GatherScatterMode.PROMISE_IN_BOUNDS)
                r = rb * LANES + l
                for cc in range(D // LANES):
                    sl = pl.ds(cc * LANES, LANES)
                    gbuf[r, sl] = gbuf[r, sl] * wv
                return carry3

            return lax.fori_loop(0, LANES, lane, carry2)

        lax.fori_loop(0, CHUNK // LANES, scale16, 0)
        pltpu.sync_copy(gbuf, acc.at[col_v.at[j]], add=True)
        return carry

    lax.fori_loop(0, NCHUNK, chunk, 0)
    plsc.subcore_barrier()

    @pl.when(s < WB_TILES)
    def _():
        pltpu.sync_copy(acc.at[pl.ds(s * WB_ROWS, WB_ROWS)],
                        out_hbm.at[pl.ds(c * N + s * WB_ROWS, WB_ROWS)])


BN = 2000
NB = N // BN
_seq = pltpu.CompilerParams(dimension_semantics=("arbitrary",))


def _tcA(deg_a, deg_b, x, W):
    def body(da_ref, db_ref, x_ref, w_ref, dis_ref, hs_ref):
        deg = da_ref[0, 0, :] + db_ref[0, 0, :] + 1.0
        dis = jnp.where(deg > 0, lax.rsqrt(jnp.maximum(deg, 1e-12)), 0.0)
        dis_ref[0, 0, :] = dis
        h = jnp.dot(x_ref[...], w_ref[...], preferred_element_type=jnp.float32)
        hs_ref[...] = h * dis[:, None]

    return pl.pallas_call(
        body,
        grid=(NB,),
        in_specs=[
            pl.BlockSpec((1, 1, BN), lambda i: (i, 0, 0)),
            pl.BlockSpec((1, 1, BN), lambda i: (i, 0, 0)),
            pl.BlockSpec((BN, D), lambda i: (i, 0)),
            pl.BlockSpec((D, D), lambda i: (0, 0)),
        ],
        out_specs=[
            pl.BlockSpec((1, 1, BN), lambda i: (i, 0, 0)),
            pl.BlockSpec((BN, D), lambda i: (i, 0)),
        ],
        out_shape=[
            jax.ShapeDtypeStruct((NB, 1, BN), jnp.float32),
            jax.ShapeDtypeStruct((N, D), jnp.float32),
        ],
        compiler_params=_seq,
    )(deg_a, deg_b, x, W)


def _tcB(sa, sb, hs0, dis, b, gamma, beta, W):
    def body(sa_ref, sb_ref, hs_ref, dis_ref, b_ref, g_ref, be_ref, w_ref,
             out_ref):
        dis_v = dis_ref[0, 0, :]
        t = (sa_ref[...] + sb_ref[...] + hs_ref[...]) * dis_v[:, None]
        t = t + b_ref[0, :][None, :]
        t = jnp.maximum(t, 0.0)
        t = t * (g_ref[0, :] / jnp.sqrt(1.0 + EPS))[None, :] + be_ref[0, :][None, :]
        h1 = jnp.dot(t, w_ref[...], preferred_element_type=jnp.float32)
        out_ref[...] = h1 * dis_v[:, None]

    return pl.pallas_call(
        body,
        grid=(NB,),
        in_specs=[
            pl.BlockSpec((BN, D), lambda i: (i, 0)),
            pl.BlockSpec((BN, D), lambda i: (i, 0)),
            pl.BlockSpec((BN, D), lambda i: (i, 0)),
            pl.BlockSpec((1, 1, BN), lambda i: (i, 0, 0)),
            pl.BlockSpec((1, D), lambda i: (0, 0)),
            pl.BlockSpec((1, D), lambda i: (0, 0)),
            pl.BlockSpec((1, D), lambda i: (0, 0)),
            pl.BlockSpec((D, D), lambda i: (0, 0)),
        ],
        out_specs=pl.BlockSpec((BN, D), lambda i: (i, 0)),
        out_shape=jax.ShapeDtypeStruct((N, D), jnp.float32),
        compiler_params=_seq,
    )(sa, sb, hs0, dis, b, gamma, beta, W)


def _tcC(sa, sb, hs1, dis, b, gamma, beta, w1, b1, w2, b2, batch):
    def body(sa_ref, sb_ref, hs_ref, dis_ref, b_ref, g_ref, be_ref,
             w1_ref, b1_ref, w2_ref, b2_ref, batch_ref, h_ref, pool_ref):
        i = pl.program_id(0)
        dis_v = dis_ref[0, 0, :]
        t = (sa_ref[...] + sb_ref[...] + hs_ref[...]) * dis_v[:, None]
        t = t + b_ref[0, :][None, :]
        t = jnp.maximum(t, 0.0)
        t = t * (g_ref[0, :] / jnp.sqrt(1.0 + EPS))[None, :] + be_ref[0, :][None, :]
        t1 = jnp.dot(t, w1_ref[...], preferred_element_type=jnp.float32)
        t1 = jnp.maximum(t1 + b1_ref[0, :][None, :], 0.0)
        ho = jnp.dot(t1, w2_ref[...], preferred_element_type=jnp.float32)
        ho = ho + b2_ref[0, :][None, :]
        h_ref[...] = ho
        seg_ids = lax.broadcasted_iota(jnp.int32, (G, BN), 0)
        M = (batch_ref[0, 0, :][None, :] == seg_ids).astype(jnp.float32)
        num = jnp.dot(M, ho, preferred_element_type=jnp.float32)
        cnt = jnp.sum(M, axis=1)
        blk = jnp.concatenate(
            [num, jnp.broadcast_to(cnt[:, None], (G, D))], axis=0)

        @pl.when(i == 0)
        def _():
            pool_ref[...] = jnp.zeros((2 * G, D), jnp.float32)

        pool_ref[...] += blk

    return pl.pallas_call(
        body,
        grid=(NB,),
        in_specs=[
            pl.BlockSpec((BN, D), lambda i: (i, 0)),
            pl.BlockSpec((BN, D), lambda i: (i, 0)),
            pl.BlockSpec((BN, D), lambda i: (i, 0)),
            pl.BlockSpec((1, 1, BN), lambda i: (i, 0, 0)),
            pl.BlockSpec((1, D), lambda i: (0, 0)),
            pl.BlockSpec((1, D), lambda i: (0, 0)),
            pl.BlockSpec((1, D), lambda i: (0, 0)),
            pl.BlockSpec((D, D), lambda i: (0, 0)),
            pl.BlockSpec((1, D), lambda i: (0, 0)),
            pl.BlockSpec((D, D), lambda i: (0, 0)),
            pl.BlockSpec((1, D), lambda i: (0, 0)),
            pl.BlockSpec((1, 1, BN), lambda i: (i, 0, 0)),
        ],
        out_specs=[
            pl.BlockSpec((BN, D), lambda i: (i, 0)),
            pl.BlockSpec((2 * G, D), lambda i: (0, 0)),
        ],
        out_shape=[
            jax.ShapeDtypeStruct((N, D), jnp.float32),
            jax.ShapeDtypeStruct((2 * G, D), jnp.float32),
        ],
        compiler_params=_seq,
    )(sa, sb, hs1, dis, b, gamma, beta, w1, b1, w2, b2, batch)


def _tcD(pool, w1, b1, w2, b2):
    def body(pool_ref, w1_ref, b1_ref, w2_ref, b2_ref, reps_ref, log_ref):
        reps = pool_ref[0:G, :] / jnp.maximum(pool_ref[G:2 * G, :], 1.0)
        reps_ref[...] = reps
        z = jnp.dot(reps, w1_ref[...], preferred_element_type=jnp.float32)
        z = jnp.maximum(z + b1_ref[0, :][None, :], 0.0)
        lg = jnp.dot(z, w2_ref[...], preferred_element_type=jnp.float32)
        log_ref[...] = lg + b2_ref[0, :][None, :]

    return pl.pallas_call(
        body,
        out_shape=[
            jax.ShapeDtypeStruct((G, D), jnp.float32),
            jax.ShapeDtypeStruct((G, OUT), jnp.float32),
        ],
    )(pool, w1, b1, w2, b2)


def kernel(x, edge_index, edge_attr, batch, W_gc0, b_gc0, gamma0, beta0,
           W_gc1, b_gc1, gamma1, beta1, lin1_W, lin1_b, lin2_W, lin2_b,
           cls1_W, cls1_b, cls2_W, cls2_b):
    ew = edge_attr[:, 0]
    row2 = edge_index[0].reshape(E // CHUNK, CHUNK)
    col2 = edge_index[1].reshape(E // CHUNK, CHUNK)
    ew2 = ew.reshape(E // CHUNK, CHUNK)

    deg2 = _deg_kernel(edge_index[1], ew).reshape(NC, N)
    dis3, hs0 = _tcA(deg2[0].reshape(NB, 1, BN), deg2[1].reshape(NB, 1, BN),
                     x, W_gc0)
    S0 = _scat_kernel(hs0, row2, col2, ew2)
    hs1 = _tcB(S0[:N], S0[N:], hs0, dis3, b_gc0.reshape(1, D),
               gamma0.reshape(1, D), beta0.reshape(1, D), W_gc1)
    S1 = _scat_kernel(hs1, row2, col2, ew2)
    hout, pool = _tcC(S1[:N], S1[N:], hs1, dis3, b_gc1.reshape(1, D),
                      gamma1.reshape(1, D), beta1.reshape(1, D),
                      lin1_W, lin1_b.reshape(1, D), lin2_W,
                      lin2_b.reshape(1, D), batch.reshape(NB, 1, BN))
    reps, logits = _tcD(pool, cls1_W, cls1_b.reshape(1, D),
                        cls2_W, cls2_b.reshape(1, OUT))
    return (hout, reps, logits)


# speedup vs baseline: 24.9181x; 1.6982x over previous
import functools

import jax
import jax.numpy as jnp
from jax import lax
from jax.experimental import pallas as pl
from jax.experimental.pallas import tpu as pltpu
from jax.experimental.pallas import tpu_sc as plsc

N = 10000
E = 320000
D = 128
OUT = 16
G = 64
EPS = 1e-5

NC, NS = 2, 16
NW = NC * NS
EW_PER = E // NW
CHUNK = 80
NCHUNK = EW_PER // CHUNK
WB_TILES = 10
WB_ROWS = N // WB_TILES
LANES = 16

_mesh = plsc.VectorSubcoreMesh(core_axis_name="c", subcore_axis_name="s",
                               num_cores=NC, num_subcores=NS)


_sc_params = pltpu.CompilerParams(use_tc_tiling_on_sc=False)


@functools.partial(
    pl.kernel,
    out_type=jax.ShapeDtypeStruct((NC * N,), jnp.float32),
    mesh=_mesh,
    scratch_types=[
        pltpu.VMEM((NCHUNK, CHUNK), jnp.int32),
        pltpu.VMEM((NCHUNK, CHUNK), jnp.float32),
        pltpu.VMEM((2000,), jnp.float32),
        pltpu.VMEM((NCHUNK + 3,), jnp.int32),
        pltpu.VMEM_SHARED((N,), jnp.float32),
        pltpu.SemaphoreType.DMA,
    ],
    compiler_params=_sc_params,
)
def _deg_kernel(col_hbm, ew_hbm, out_hbm, col_v, ew_v, zbuf, idx_v, acc,
                dsem):
    c = lax.axis_index("c")
    s = lax.axis_index("s")
    wid = s * NC + c

    def ifill(i, carry):
        iv = (lax.broadcasted_iota(jnp.int32, (LANES,), 0)
              + (wid * NCHUNK + i * LANES))
        idx_v[pl.ds(i * LANES, LANES)] = jnp.minimum(iv, NW * NCHUNK - 1)
        return carry

    lax.fori_loop(0, (NCHUNK + 3) // LANES, ifill, 0)
    idx125 = idx_v.at[pl.ds(0, NCHUNK)]
    pltpu.async_copy(col_hbm.at[idx125], col_v, dsem).wait()
    pltpu.async_copy(ew_hbm.at[idx125], ew_v, dsem).wait()

    def zf(i, carry):
        zbuf[pl.ds(i * LANES, LANES)] = jnp.zeros((LANES,), jnp.float32)
        return carry

    lax.fori_loop(0, 2000 // LANES, zf, 0)

    @pl.when(s < 5)
    def _():
        pltpu.sync_copy(zbuf, acc.at[pl.ds(s * 2000, 2000)])

    plsc.subcore_barrier()

    def burst(b, carry):
        def fire(k, carry2):
            pltpu.async_copy(ew_v.at[b * 25 + k], acc.at[col_v.at[b * 25 + k]],
                             dsem, add=True)
            return carry2

        lax.fori_loop(0, 25, fire, 0)

        def drain(k, carry2):
            pltpu.make_async_copy(ew_v.at[0], acc.at[col_v.at[0]],
                                  dsem).wait()
            return carry2

        lax.fori_loop(0, 25, drain, 0)
        return carry

    lax.fori_loop(0, NCHUNK // 25, burst, 0)
    plsc.subcore_barrier()

    @pl.when(s < 5)
    def _():
        pltpu.sync_copy(acc.at[pl.ds(s * 2000, 2000)], zbuf)
        pltpu.sync_copy(zbuf, out_hbm.at[pl.ds(c * N + s * 2000, 2000)])


@functools.partial(
    pl.kernel,
    out_type=jax.ShapeDtypeStruct((NC * N, D), jnp.float32),
    mesh=_mesh,
    scratch_types=[
        pltpu.VMEM((NCHUNK, CHUNK), jnp.int32),
        pltpu.VMEM((NCHUNK, CHUNK), jnp.int32),
        pltpu.VMEM((NCHUNK, CHUNK), jnp.float32),
        pltpu.VMEM((CHUNK, D), jnp.float32),
        pltpu.VMEM((CHUNK, D), jnp.float32),
        pltpu.VMEM((NCHUNK + 3,), jnp.int32),
        pltpu.VMEM_SHARED((N, D), jnp.float32),
        pltpu.SemaphoreType.DMA,
        pltpu.SemaphoreType.DMA,
        pltpu.SemaphoreType.DMA,
        pltpu.SemaphoreType.DMA,
    ],
    compiler_params=_sc_params,
)
def _scat_kernel(hs_hbm, row_hbm, col_hbm, ew_hbm, out_hbm,
                 row_v, col_v, ew_v, gbufA, gbufB, idx_v, acc,
                 gsemA, gsemB, ssemA, ssemB):
    c = lax.axis_index("c")
    s = lax.axis_index("s")
    wid = s * NC + c

    def ifill(i, carry):
        iv = (lax.broadcasted_iota(jnp.int32, (LANES,), 0)
              + (wid * NCHUNK + i * LANES))
        idx_v[pl.ds(i * LANES, LANES)] = jnp.minimum(iv, NW * NCHUNK - 1)
        return carry

    lax.fori_loop(0, (NCHUNK + 3) // LANES, ifill, 0)
    idx125 = idx_v.at[pl.ds(0, NCHUNK)]
    pltpu.async_copy(row_hbm.at[idx125], row_v, gsemA).wait()
    pltpu.async_copy(col_hbm.at[idx125], col_v, gsemA).wait()
    pltpu.async_copy(ew_hbm.at[idx125], ew_v, gsemA).wait()

    def zf(r, carry):
        for cc in range(D // LANES):
            gbufA[r, pl.ds(cc * LANES, LANES)] = jnp.zeros((LANES,),
                                                           jnp.float32)
        return carry

    lax.fori_loop(0, CHUNK, zf, 0)

    @pl.when(s < WB_TILES)
    def _():
        def zc(m, carry):
            pltpu.sync_copy(gbufA.at[pl.ds(0, 40)],
                            acc.at[pl.ds(s * WB_ROWS + m * 40, 40)])
            return carry

        lax.fori_loop(0, WB_ROWS // 40, zc, 0)

    plsc.subcore_barrier()

    dn = lax.GatherDimensionNumbers(
        offset_dims=(), collapsed_slice_dims=(0,), start_index_map=(0,))

    def scale(buf, j):
        def scale16(rb, carry):
            wch = ew_v[j, pl.ds(rb * LANES, LANES)]
            for l in range(LANES):
                wv = lax.gather(wch, jnp.full((LANES, 1), l, jnp.int32), dn,
                                slice_sizes=(1,),
                                mode=lax.GatherScatterMode.PROMISE_IN_BOUNDS)
                r = rb * LANES + l
                for cc in range(D // LANES):
                    sl = pl.ds(cc * LANES, LANES)
                    buf[r, sl] = buf[r, sl] * wv
            return carry

        lax.fori_loop(0, CHUNK // LANES, scale16, 0)

    def gather_start(j, buf, sem):
        return pltpu.async_copy(hs_hbm.at[row_v.at[j]], buf, sem)

    def gather_wait(buf, sem):
        pltpu.make_async_copy(hs_hbm.at[row_v.at[0]], buf, sem).wait()

    def scat_start(j, buf, sem):
        pltpu.async_copy(buf, acc.at[col_v.at[j]], sem, add=True)

    def scat_wait(buf, sem):
        pltpu.make_async_copy(buf, acc.at[col_v.at[0]], sem).wait()

    gather_start(0, gbufA, gsemA)

    def body(m, carry):
        jA = 2 * m
        jB = 2 * m + 1

        @pl.when(m > 0)
        def _():
            scat_wait(gbufB, ssemB)

        gather_start(jB, gbufB, gsemB)
        gather_wait(gbufA, gsemA)
        scale(gbufA, jA)
        scat_start(jA, gbufA, ssemA)
        gather_wait(gbufB, gsemB)
        scale(gbufB, jB)
        scat_wait(gbufA, ssemA)
        gather_start(jA + 2, gbufA, gsemA)
        scat_start(jB, gbufB, ssemB)
        return carry

    lax.fori_loop(0, (NCHUNK - 1) // 2, body, 0)

    gather_wait(gbufA, gsemA)
    scale(gbufA, NCHUNK - 1)
    scat_wait(gbufB, ssemB)
    pltpu.sync_copy(gbufA, acc.at[col_v.at[NCHUNK - 1]], add=True)

    plsc.subcore_barrier()

    @pl.when(s < WB_TILES)
    def _():
        pltpu.sync_copy(acc.at[pl.ds(s * WB_ROWS, WB_ROWS)],
                        out_hbm.at[pl.ds(c * N + s * WB_ROWS, WB_ROWS)])


BN = 2000
NB = N // BN
_seq = pltpu.CompilerParams(dimension_semantics=("arbitrary",))


def _tcA(deg_a, deg_b, x, W):
    def body(da_ref, db_ref, x_ref, w_ref, dis_ref, hs_ref):
        deg = da_ref[0, 0, :] + db_ref[0, 0, :] + 1.0
        dis = jnp.where(deg > 0, lax.rsqrt(jnp.maximum(deg, 1e-12)), 0.0)
        dis_ref[0, 0, :] = dis
        h = jnp.dot(x_ref[...], w_ref[...], preferred_element_type=jnp.float32)
        hs_ref[...] = h * dis[:, None]

    return pl.pallas_call(
        body,
        grid=(NB,),
        in_specs=[
            pl.BlockSpec((1, 1, BN), lambda i: (i, 0, 0)),
            pl.BlockSpec((1, 1, BN), lambda i: (i, 0, 0)),
            pl.BlockSpec((BN, D), lambda i: (i, 0)),
            pl.BlockSpec((D, D), lambda i: (0, 0)),
        ],
        out_specs=[
            pl.BlockSpec((1, 1, BN), lambda i: (i, 0, 0)),
            pl.BlockSpec((BN, D), lambda i: (i, 0)),
        ],
        out_shape=[
            jax.ShapeDtypeStruct((NB, 1, BN), jnp.float32),
            jax.ShapeDtypeStruct((N, D), jnp.float32),
        ],
        compiler_params=_seq,
    )(deg_a, deg_b, x, W)


def _tcB(sa, sb, hs0, dis, b, gamma, beta, W):
    def body(sa_ref, sb_ref, hs_ref, dis_ref, b_ref, g_ref, be_ref, w_ref,
             out_ref):
        dis_v = dis_ref[0, 0, :]
        t = (sa_ref[...] + sb_ref[...] + hs_ref[...]) * dis_v[:, None]
        t = t + b_ref[0, :][None, :]
        t = jnp.maximum(t, 0.0)
        t = t * (g_ref[0, :] / jnp.sqrt(1.0 + EPS))[None, :] + be_ref[0, :][None, :]
        h1 = jnp.dot(t, w_ref[...], preferred_element_type=jnp.float32)
        out_ref[...] = h1 * dis_v[:, None]

    return pl.pallas_call(
        body,
        grid=(NB,),
        in_specs=[
            pl.BlockSpec((BN, D), lambda i: (i, 0)),
            pl.BlockSpec((BN, D), lambda i: (i, 0)),
            pl.BlockSpec((BN, D), lambda i: (i, 0)),
            pl.BlockSpec((1, 1, BN), lambda i: (i, 0, 0)),
            pl.BlockSpec((1, D), lambda i: (0, 0)),
            pl.BlockSpec((1, D), lambda i: (0, 0)),
            pl.BlockSpec((1, D), lambda i: (0, 0)),
            pl.BlockSpec((D, D), lambda i: (0, 0)),
        ],
        out_specs=pl.BlockSpec((BN, D), lambda i: (i, 0)),
        out_shape=jax.ShapeDtypeStruct((N, D), jnp.float32),
        compiler_params=_seq,
    )(sa, sb, hs0, dis, b, gamma, beta, W)


def _tcC(sa, sb, hs1, dis, b, gamma, beta, w1, b1, w2, b2, batch):
    def body(sa_ref, sb_ref, hs_ref, dis_ref, b_ref, g_ref, be_ref,
             w1_ref, b1_ref, w2_ref, b2_ref, batch_ref, h_ref, pool_ref):
        i = pl.program_id(0)
        dis_v = dis_ref[0, 0, :]
        t = (sa_ref[...] + sb_ref[...] + hs_ref[...]) * dis_v[:, None]
        t = t + b_ref[0, :][None, :]
        t = jnp.maximum(t, 0.0)
        t = t * (g_ref[0, :] / jnp.sqrt(1.0 + EPS))[None, :] + be_ref[0, :][None, :]
        t1 = jnp.dot(t, w1_ref[...], preferred_element_type=jnp.float32)
        t1 = jnp.maximum(t1 + b1_ref[0, :][None, :], 0.0)
        ho = jnp.dot(t1, w2_ref[...], preferred_element_type=jnp.float32)
        ho = ho + b2_ref[0, :][None, :]
        h_ref[...] = ho
        seg_ids = lax.broadcasted_iota(jnp.int32, (G, BN), 0)
        M = (batch_ref[0, 0, :][None, :] == seg_ids).astype(jnp.float32)
        num = jnp.dot(M, ho, preferred_element_type=jnp.float32)
        cnt = jnp.sum(M, axis=1)
        blk = jnp.concatenate(
            [num, jnp.broadcast_to(cnt[:, None], (G, D))], axis=0)

        @pl.when(i == 0)
        def _():
            pool_ref[...] = jnp.zeros((2 * G, D), jnp.float32)

        pool_ref[...] += blk

    return pl.pallas_call(
        body,
        grid=(NB,),
        in_specs=[
            pl.BlockSpec((BN, D), lambda i: (i, 0)),
            pl.BlockSpec((BN, D), lambda i: (i, 0)),
            pl.BlockSpec((BN, D), lambda i: (i, 0)),
            pl.BlockSpec((1, 1, BN), lambda i: (i, 0, 0)),
            pl.BlockSpec((1, D), lambda i: (0, 0)),
            pl.BlockSpec((1, D), lambda i: (0, 0)),
            pl.BlockSpec((1, D), lambda i: (0, 0)),
            pl.BlockSpec((D, D), lambda i: (0, 0)),
            pl.BlockSpec((1, D), lambda i: (0, 0)),
            pl.BlockSpec((D, D), lambda i: (0, 0)),
            pl.BlockSpec((1, D), lambda i: (0, 0)),
            pl.BlockSpec((1, 1, BN), lambda i: (i, 0, 0)),
        ],
        out_specs=[
            pl.BlockSpec((BN, D), lambda i: (i, 0)),
            pl.BlockSpec((2 * G, D), lambda i: (0, 0)),
        ],
        out_shape=[
            jax.ShapeDtypeStruct((N, D), jnp.float32),
            jax.ShapeDtypeStruct((2 * G, D), jnp.float32),
        ],
        compiler_params=_seq,
    )(sa, sb, hs1, dis, b, gamma, beta, w1, b1, w2, b2, batch)


def _tcD(pool, w1, b1, w2, b2):
    def body(pool_ref, w1_ref, b1_ref, w2_ref, b2_ref, reps_ref, log_ref):
        reps = pool_ref[0:G, :] / jnp.maximum(pool_ref[G:2 * G, :], 1.0)
        reps_ref[...] = reps
        z = jnp.dot(reps, w1_ref[...], preferred_element_type=jnp.float32)
        z = jnp.maximum(z + b1_ref[0, :][None, :], 0.0)
        lg = jnp.dot(z, w2_ref[...], preferred_element_type=jnp.float32)
        log_ref[...] = lg + b2_ref[0, :][None, :]

    return pl.pallas_call(
        body,
        out_shape=[
            jax.ShapeDtypeStruct((G, D), jnp.float32),
            jax.ShapeDtypeStruct((G, OUT), jnp.float32),
        ],
    )(pool, w1, b1, w2, b2)


def kernel(x, edge_index, edge_attr, batch, W_gc0, b_gc0, gamma0, beta0,
           W_gc1, b_gc1, gamma1, beta1, lin1_W, lin1_b, lin2_W, lin2_b,
           cls1_W, cls1_b, cls2_W, cls2_b):
    ew = edge_attr[:, 0]
    row2 = edge_index[0].reshape(E // CHUNK, CHUNK)
    col2 = edge_index[1].reshape(E // CHUNK, CHUNK)
    ew2 = ew.reshape(E // CHUNK, CHUNK)

    deg2 = _deg_kernel(col2, ew2).reshape(NC, N)
    dis3, hs0 = _tcA(deg2[0].reshape(NB, 1, BN), deg2[1].reshape(NB, 1, BN),
                     x, W_gc0)
    S0 = _scat_kernel(hs0, row2, col2, ew2)
    hs1 = _tcB(S0[:N], S0[N:], hs0, dis3, b_gc0.reshape(1, D),
               gamma0.reshape(1, D), beta0.reshape(1, D), W_gc1)
    S1 = _scat_kernel(hs1, row2, col2, ew2)
    hout, pool = _tcC(S1[:N], S1[N:], hs1, dis3, b_gc1.reshape(1, D),
                      gamma1.reshape(1, D), beta1.reshape(1, D),
                      lin1_W, lin1_b.reshape(1, D), lin2_W,
                      lin2_b.reshape(1, D), batch.reshape(NB, 1, BN))
    reps, logits = _tcD(pool, cls1_W, cls1_b.reshape(1, D),
                        cls2_W, cls2_b.reshape(1, OUT))
    return (hout, reps, logits)
